# baseline (speedup 1.0000x reference)
import jax
import jax.numpy as jnp
from jax.experimental import pallas as pl
from jax.experimental.pallas import tpu as pltpu

K = 64
LANES = 128
CHUNK = 512


def _ce_small(v, d, size, flip):
    rows = v.shape[0]
    ii = jax.lax.broadcasted_iota(jnp.int32, v.shape, 0)
    low_bit = (ii & d) == 0
    asc_blk = (ii & size) != 0
    partner = jnp.where(low_bit, pltpu.roll(v, rows - d, 0), pltpu.roll(v, d, 0))
    want_max = (low_bit != asc_blk) != flip
    return jnp.where(want_max, jnp.maximum(v, partner), jnp.minimum(v, partner))


def _ce_big(v, d, size, flip):
    lanes = v.shape[1]
    g = v.reshape(-1, 2 * d, lanes)
    a = g[:, :d, :]
    b = g[:, d:, :]
    hi = jnp.maximum(a, b)
    lo = jnp.minimum(a, b)
    m = size // (2 * d)
    gi = jax.lax.broadcasted_iota(jnp.int32, (g.shape[0], 1, 1), 0)
    asc = ((gi & m) != 0) != flip
    top = jnp.where(asc, lo, hi)
    bot = jnp.where(asc, hi, lo)
    return jnp.concatenate([top, bot], axis=1).reshape(-1, lanes)


def _ce(v, d, size, flip=False):
    if d >= 8:
        return _ce_big(v, d, size, flip)
    return _ce_small(v, d, size, flip)


def _resort64(v, flip):
    for d in (32, 16, 8, 4, 2, 1):
        v = _ce(v, d, K, flip)
    return v


def _chunk_topk_asc(v):
    p1_flip = v.shape[0] == K
    size = 2
    while size <= K:
        d = size // 2
        while d >= 1:
            v = _ce(v, d, size, p1_flip)
            d //= 2
        size *= 2
    while v.shape[0] > K:
        g = v.reshape(-1, 2 * K, v.shape[1])
        v = jnp.maximum(g[:, :K, :], g[:, K:, :]).reshape(-1, v.shape[1])
        v = _resort64(v, flip=(v.shape[0] == K))
    return v


def _ce_v(v, dv, sizev, flip):
    return _ce_big(v, 8 * dv, 8 * sizev, flip)


def _chunk_runs_asc(v):
    size = 2
    while size <= K:
        d = size // 2
        while d >= 1:
            v = _ce_v(v, d, size, True)
            d //= 2
        size *= 2
    while v.shape[0] > 8 * K:
        g = v.reshape(-1, 16 * K, v.shape[1])
        v = jnp.maximum(g[:, : 8 * K, :], g[:, 8 * K :, :]).reshape(-1, v.shape[1])
        for d in (32, 16, 8, 4, 2, 1):
            v = _ce_v(v, d, K, True)
    return v


def _final_topk_desc(v):
    size = 2
    while size <= K:
        d = size // 2
        while d >= 1:
            v = _ce(v, d, size)
            d //= 2
        size *= 2
    while v.shape[0] > K:
        g = v.reshape(-1, 2 * K, v.shape[1])
        v = jnp.maximum(g[:, :K, :], g[:, K:, :]).reshape(-1, v.shape[1])
        v = _resort64(v, flip=False)
    return v


def _vregrev(v):
    n = v.shape[0] // 8
    return jnp.concatenate([v[8 * i : 8 * i + 8] for i in reversed(range(n))], 0)


def _final_topk_desc_v2(v):
    for t in (4, 2, 1):
        u = pltpu.roll(_vregrev(v), t, 0)
        v = jnp.maximum(v, u)
        for d in (32, 16, 8, 4, 2, 1):
            v = _ce_v(v, d, K, False)
    return v.reshape(K, 8, v.shape[1])[:, 7, :]


def _kmax_body(x_ref, o_ref, acc_ref):
    s_idx = pl.program_id(2)
    n_s = pl.num_programs(2)
    chunk = _chunk_runs_asc(x_ref[0])

    @pl.when(s_idx == 0)
    def _init():
        acc_ref[...] = jnp.full(acc_ref.shape, -jnp.inf, jnp.float32)

    merged = jnp.maximum(acc_ref[...], chunk)
    for d in (32, 16, 8, 4, 2, 1):
        merged = _ce_v(merged, d, K, False)
    acc_ref[...] = merged

    @pl.when(s_idx == n_s - 1)
    def _finish():
        o_ref[0] = _final_topk_desc_v2(merged)


def _kmax_body_small(x_ref, o_ref):
    o_ref[0] = _final_topk_desc(x_ref[0])


def kernel(inputs):
    b, s, c = inputs.shape
    if s < CHUNK:
        return pl.pallas_call(
            _kmax_body_small,
            grid=(b, c // LANES),
            in_specs=[pl.BlockSpec((1, s, LANES), lambda i, j: (i, 0, j))],
            out_specs=pl.BlockSpec((1, K, LANES), lambda i, j: (i, 0, j)),
            out_shape=jax.ShapeDtypeStruct((b, K, c), jnp.float32),
        )(inputs)
    grid = (b, c // LANES, s // CHUNK)
    out = pl.pallas_call(
        _kmax_body,
        grid=grid,
        in_specs=[pl.BlockSpec((1, CHUNK, LANES), lambda i, j, k: (i, k, j))],
        out_specs=pl.BlockSpec((1, K, LANES), lambda i, j, k: (i, 0, j)),
        out_shape=jax.ShapeDtypeStruct((b, K, c), jnp.float32),
        scratch_shapes=[pltpu.VMEM((8 * K, LANES), jnp.float32)],
    )(inputs)
    return out


# speedup vs baseline: 1.4314x; 1.4314x over previous
import jax
import jax.numpy as jnp
from jax.experimental import pallas as pl
from jax.experimental.pallas import tpu as pltpu

K = 64
LANES = 128
CHUNK = 1024


def _ce_small(v, d, size, flip):
    rows = v.shape[0]
    ii = jax.lax.broadcasted_iota(jnp.int32, v.shape, 0)
    low_bit = (ii & d) == 0
    asc_blk = (ii & size) != 0
    partner = jnp.where(low_bit, pltpu.roll(v, rows - d, 0), pltpu.roll(v, d, 0))
    want_max = (low_bit != asc_blk) != flip
    return jnp.where(want_max, jnp.maximum(v, partner), jnp.minimum(v, partner))


def _ce_big(v, d, size, flip):
    lanes = v.shape[1]
    g = v.reshape(-1, 2 * d, lanes)
    a = g[:, :d, :]
    b = g[:, d:, :]
    hi = jnp.maximum(a, b)
    lo = jnp.minimum(a, b)
    m = size // (2 * d)
    gi = jax.lax.broadcasted_iota(jnp.int32, (g.shape[0], 1, 1), 0)
    asc = ((gi & m) != 0) != flip
    top = jnp.where(asc, lo, hi)
    bot = jnp.where(asc, hi, lo)
    return jnp.concatenate([top, bot], axis=1).reshape(-1, lanes)


def _ce(v, d, size, flip=False):
    if d >= 8:
        return _ce_big(v, d, size, flip)
    return _ce_small(v, d, size, flip)


def _resort64(v, flip):
    for d in (32, 16, 8, 4, 2, 1):
        v = _ce(v, d, K, flip)
    return v


def _chunk_topk_asc(v):
    p1_flip = v.shape[0] == K
    size = 2
    while size <= K:
        d = size // 2
        while d >= 1:
            v = _ce(v, d, size, p1_flip)
            d //= 2
        size *= 2
    while v.shape[0] > K:
        g = v.reshape(-1, 2 * K, v.shape[1])
        v = jnp.maximum(g[:, :K, :], g[:, K:, :]).reshape(-1, v.shape[1])
        v = _resort64(v, flip=(v.shape[0] == K))
    return v


def _ce_v(v, dv, sizev, flip):
    return _ce_big(v, 8 * dv, 8 * sizev, flip)


def _chunk_runs_asc(v):
    size = 2
    while size <= K:
        d = size // 2
        while d >= 1:
            v = _ce_v(v, d, size, True)
            d //= 2
        size *= 2
    while v.shape[0] > 8 * K:
        g = v.reshape(-1, 16 * K, v.shape[1])
        v = jnp.maximum(g[:, : 8 * K, :], g[:, 8 * K :, :]).reshape(-1, v.shape[1])
        for d in (32, 16, 8, 4, 2, 1):
            v = _ce_v(v, d, K, True)
    return v


def _final_topk_desc(v):
    size = 2
    while size <= K:
        d = size // 2
        while d >= 1:
            v = _ce(v, d, size)
            d //= 2
        size *= 2
    while v.shape[0] > K:
        g = v.reshape(-1, 2 * K, v.shape[1])
        v = jnp.maximum(g[:, :K, :], g[:, K:, :]).reshape(-1, v.shape[1])
        v = _resort64(v, flip=False)
    return v


def _vregrev(v):
    n = v.shape[0] // 8
    return jnp.concatenate([v[8 * i : 8 * i + 8] for i in reversed(range(n))], 0)


def _final_topk_desc_v2(v):
    for t in (4, 2, 1):
        u = pltpu.roll(_vregrev(v), t, 0)
        v = jnp.maximum(v, u)
        for d in (32, 16, 8, 4, 2, 1):
            v = _ce_v(v, d, K, False)
    return v.reshape(K, 8, v.shape[1])[:, 7, :]


def _kmax_body(x_ref, o_ref, acc_ref):
    s_idx = pl.program_id(2)
    n_s = pl.num_programs(2)

    @pl.when(s_idx == 0)
    def _init():
        acc_ref[...] = jnp.full(acc_ref.shape, -jnp.inf, jnp.float32)

    merged = acc_ref[...]
    for h in range(x_ref.shape[1] // 512):
        sub = _chunk_runs_asc(x_ref[0, 512 * h : 512 * (h + 1), :])
        merged = jnp.maximum(merged, sub)
        for d in (32, 16, 8, 4, 2, 1):
            merged = _ce_v(merged, d, K, False)
    acc_ref[...] = merged

    @pl.when(s_idx == n_s - 1)
    def _finish():
        o_ref[0] = _final_topk_desc_v2(merged)


def _kmax_body_small(x_ref, o_ref):
    o_ref[0] = _final_topk_desc(x_ref[0])


def kernel(inputs):
    b, s, c = inputs.shape
    if s < CHUNK:
        return pl.pallas_call(
            _kmax_body_small,
            grid=(b, c // LANES),
            in_specs=[pl.BlockSpec((1, s, LANES), lambda i, j: (i, 0, j))],
            out_specs=pl.BlockSpec((1, K, LANES), lambda i, j: (i, 0, j)),
            out_shape=jax.ShapeDtypeStruct((b, K, c), jnp.float32),
        )(inputs)
    grid = (b, c // LANES, s // CHUNK)
    out = pl.pallas_call(
        _kmax_body,
        grid=grid,
        in_specs=[pl.BlockSpec((1, CHUNK, LANES), lambda i, j, k: (i, k, j))],
        out_specs=pl.BlockSpec((1, K, LANES), lambda i, j, k: (i, 0, j)),
        out_shape=jax.ShapeDtypeStruct((b, K, c), jnp.float32),
        scratch_shapes=[pltpu.VMEM((8 * K, LANES), jnp.float32)],
    )(inputs)
    return out


# speedup vs baseline: 1.7430x; 1.2177x over previous
import jax
import jax.numpy as jnp
from jax.experimental import pallas as pl
from jax.experimental.pallas import tpu as pltpu

K = 64
LANES = 128
CHUNK = 2048


def _ce_small(v, d, size, flip):
    rows = v.shape[0]
    ii = jax.lax.broadcasted_iota(jnp.int32, v.shape, 0)
    low_bit = (ii & d) == 0
    asc_blk = (ii & size) != 0
    partner = jnp.where(low_bit, pltpu.roll(v, rows - d, 0), pltpu.roll(v, d, 0))
    want_max = (low_bit != asc_blk) != flip
    return jnp.where(want_max, jnp.maximum(v, partner), jnp.minimum(v, partner))


def _ce_big(v, d, size, flip):
    lanes = v.shape[1]
    g = v.reshape(-1, 2 * d, lanes)
    a = g[:, :d, :]
    b = g[:, d:, :]
    hi = jnp.maximum(a, b)
    lo = jnp.minimum(a, b)
    m = size // (2 * d)
    gi = jax.lax.broadcasted_iota(jnp.int32, (g.shape[0], 1, 1), 0)
    asc = ((gi & m) != 0) != flip
    top = jnp.where(asc, lo, hi)
    bot = jnp.where(asc, hi, lo)
    return jnp.concatenate([top, bot], axis=1).reshape(-1, lanes)


def _ce(v, d, size, flip=False):
    if d >= 8:
        return _ce_big(v, d, size, flip)
    return _ce_small(v, d, size, flip)


def _resort64(v, flip):
    for d in (32, 16, 8, 4, 2, 1):
        v = _ce(v, d, K, flip)
    return v


def _chunk_topk_asc(v):
    p1_flip = v.shape[0] == K
    size = 2
    while size <= K:
        d = size // 2
        while d >= 1:
            v = _ce(v, d, size, p1_flip)
            d //= 2
        size *= 2
    while v.shape[0] > K:
        g = v.reshape(-1, 2 * K, v.shape[1])
        v = jnp.maximum(g[:, :K, :], g[:, K:, :]).reshape(-1, v.shape[1])
        v = _resort64(v, flip=(v.shape[0] == K))
    return v


def _ce_v(v, dv, sizev, flip):
    return _ce_big(v, 8 * dv, 8 * sizev, flip)


def _chunk_runs_asc(v):
    size = 2
    while size <= K:
        d = size // 2
        while d >= 1:
            v = _ce_v(v, d, size, True)
            d //= 2
        size *= 2
    while v.shape[0] > 8 * K:
        g = v.reshape(-1, 16 * K, v.shape[1])
        v = jnp.maximum(g[:, : 8 * K, :], g[:, 8 * K :, :]).reshape(-1, v.shape[1])
        for d in (32, 16, 8, 4, 2, 1):
            v = _ce_v(v, d, K, True)
    return v


def _final_topk_desc(v):
    size = 2
    while size <= K:
        d = size // 2
        while d >= 1:
            v = _ce(v, d, size)
            d //= 2
        size *= 2
    while v.shape[0] > K:
        g = v.reshape(-1, 2 * K, v.shape[1])
        v = jnp.maximum(g[:, :K, :], g[:, K:, :]).reshape(-1, v.shape[1])
        v = _resort64(v, flip=False)
    return v


def _vregrev(v):
    n = v.shape[0] // 8
    return jnp.concatenate([v[8 * i : 8 * i + 8] for i in reversed(range(n))], 0)


def _final_topk_desc_v2(v):
    for t in (4, 2, 1):
        u = pltpu.roll(_vregrev(v), t, 0)
        v = jnp.maximum(v, u)
        for d in (32, 16, 8, 4, 2, 1):
            v = _ce_v(v, d, K, False)
    return v.reshape(K, 8, v.shape[1])[:, 7, :]


def _kmax_body(x_ref, o_ref, acc_ref):
    s_idx = pl.program_id(2)
    n_s = pl.num_programs(2)

    @pl.when(s_idx == 0)
    def _init():
        acc_ref[...] = jnp.full(acc_ref.shape, -jnp.inf, jnp.float32)

    merged = acc_ref[...]
    for h in range(x_ref.shape[1] // 512):
        sub = _chunk_runs_asc(x_ref[0, 512 * h : 512 * (h + 1), :])
        merged = jnp.maximum(merged, sub)
        for d in (32, 16, 8, 4, 2, 1):
            merged = _ce_v(merged, d, K, False)
    acc_ref[...] = merged

    @pl.when(s_idx == n_s - 1)
    def _finish():
        o_ref[0] = _final_topk_desc_v2(merged)


def _kmax_body_small(x_ref, o_ref):
    o_ref[0] = _final_topk_desc(x_ref[0])


def kernel(inputs):
    b, s, c = inputs.shape
    if s < CHUNK:
        return pl.pallas_call(
            _kmax_body_small,
            grid=(b, c // LANES),
            in_specs=[pl.BlockSpec((1, s, LANES), lambda i, j: (i, 0, j))],
            out_specs=pl.BlockSpec((1, K, LANES), lambda i, j: (i, 0, j)),
            out_shape=jax.ShapeDtypeStruct((b, K, c), jnp.float32),
        )(inputs)
    grid = (b, c // LANES, s // CHUNK)
    out = pl.pallas_call(
        _kmax_body,
        grid=grid,
        in_specs=[pl.BlockSpec((1, CHUNK, LANES), lambda i, j, k: (i, k, j))],
        out_specs=pl.BlockSpec((1, K, LANES), lambda i, j, k: (i, 0, j)),
        out_shape=jax.ShapeDtypeStruct((b, K, c), jnp.float32),
        scratch_shapes=[pltpu.VMEM((8 * K, LANES), jnp.float32)],
    )(inputs)
    return out


# speedup vs baseline: 1.7512x; 1.0047x over previous
import jax
import jax.numpy as jnp
from jax.experimental import pallas as pl
from jax.experimental.pallas import tpu as pltpu

K = 64
LANES = 128
CHUNK = 4096


def _ce_small(v, d, size, flip):
    rows = v.shape[0]
    ii = jax.lax.broadcasted_iota(jnp.int32, v.shape, 0)
    low_bit = (ii & d) == 0
    asc_blk = (ii & size) != 0
    partner = jnp.where(low_bit, pltpu.roll(v, rows - d, 0), pltpu.roll(v, d, 0))
    want_max = (low_bit != asc_blk) != flip
    return jnp.where(want_max, jnp.maximum(v, partner), jnp.minimum(v, partner))


def _ce_big(v, d, size, flip):
    lanes = v.shape[1]
    g = v.reshape(-1, 2 * d, lanes)
    a = g[:, :d, :]
    b = g[:, d:, :]
    hi = jnp.maximum(a, b)
    lo = jnp.minimum(a, b)
    m = size // (2 * d)
    gi = jax.lax.broadcasted_iota(jnp.int32, (g.shape[0], 1, 1), 0)
    asc = ((gi & m) != 0) != flip
    top = jnp.where(asc, lo, hi)
    bot = jnp.where(asc, hi, lo)
    return jnp.concatenate([top, bot], axis=1).reshape(-1, lanes)


def _ce(v, d, size, flip=False):
    if d >= 8:
        return _ce_big(v, d, size, flip)
    return _ce_small(v, d, size, flip)


def _resort64(v, flip):
    for d in (32, 16, 8, 4, 2, 1):
        v = _ce(v, d, K, flip)
    return v


def _chunk_topk_asc(v):
    p1_flip = v.shape[0] == K
    size = 2
    while size <= K:
        d = size // 2
        while d >= 1:
            v = _ce(v, d, size, p1_flip)
            d //= 2
        size *= 2
    while v.shape[0] > K:
        g = v.reshape(-1, 2 * K, v.shape[1])
        v = jnp.maximum(g[:, :K, :], g[:, K:, :]).reshape(-1, v.shape[1])
        v = _resort64(v, flip=(v.shape[0] == K))
    return v


def _ce_v(v, dv, sizev, flip):
    return _ce_big(v, 8 * dv, 8 * sizev, flip)


def _chunk_runs_asc(v):
    size = 2
    while size <= K:
        d = size // 2
        while d >= 1:
            v = _ce_v(v, d, size, True)
            d //= 2
        size *= 2
    while v.shape[0] > 8 * K:
        g = v.reshape(-1, 16 * K, v.shape[1])
        v = jnp.maximum(g[:, : 8 * K, :], g[:, 8 * K :, :]).reshape(-1, v.shape[1])
        for d in (32, 16, 8, 4, 2, 1):
            v = _ce_v(v, d, K, True)
    return v


def _final_topk_desc(v):
    size = 2
    while size <= K:
        d = size // 2
        while d >= 1:
            v = _ce(v, d, size)
            d //= 2
        size *= 2
    while v.shape[0] > K:
        g = v.reshape(-1, 2 * K, v.shape[1])
        v = jnp.maximum(g[:, :K, :], g[:, K:, :]).reshape(-1, v.shape[1])
        v = _resort64(v, flip=False)
    return v


def _vregrev(v):
    n = v.shape[0] // 8
    return jnp.concatenate([v[8 * i : 8 * i + 8] for i in reversed(range(n))], 0)


def _final_topk_desc_v2(v):
    for t in (4, 2, 1):
        u = pltpu.roll(_vregrev(v), t, 0)
        v = jnp.maximum(v, u)
        for d in (32, 16, 8, 4, 2, 1):
            v = _ce_v(v, d, K, False)
    return v.reshape(K, 8, v.shape[1])[:, 7, :]


def _kmax_body(x_ref, o_ref, acc_ref):
    s_idx = pl.program_id(2)
    n_s = pl.num_programs(2)

    @pl.when(s_idx == 0)
    def _init():
        acc_ref[...] = jnp.full(acc_ref.shape, -jnp.inf, jnp.float32)

    merged = acc_ref[...]
    for h in range(x_ref.shape[1] // 512):
        sub = _chunk_runs_asc(x_ref[0, 512 * h : 512 * (h + 1), :])
        merged = jnp.maximum(merged, sub)
        for d in (32, 16, 8, 4, 2, 1):
            merged = _ce_v(merged, d, K, False)
    acc_ref[...] = merged

    @pl.when(s_idx == n_s - 1)
    def _finish():
        o_ref[0] = _final_topk_desc_v2(merged)


def _kmax_body_small(x_ref, o_ref):
    o_ref[0] = _final_topk_desc(x_ref[0])


def kernel(inputs):
    b, s, c = inputs.shape
    if s < CHUNK:
        return pl.pallas_call(
            _kmax_body_small,
            grid=(b, c // LANES),
            in_specs=[pl.BlockSpec((1, s, LANES), lambda i, j: (i, 0, j))],
            out_specs=pl.BlockSpec((1, K, LANES), lambda i, j: (i, 0, j)),
            out_shape=jax.ShapeDtypeStruct((b, K, c), jnp.float32),
        )(inputs)
    grid = (b, c // LANES, s // CHUNK)
    out = pl.pallas_call(
        _kmax_body,
        grid=grid,
        in_specs=[pl.BlockSpec((1, CHUNK, LANES), lambda i, j, k: (i, k, j))],
        out_specs=pl.BlockSpec((1, K, LANES), lambda i, j, k: (i, 0, j)),
        out_shape=jax.ShapeDtypeStruct((b, K, c), jnp.float32),
        scratch_shapes=[pltpu.VMEM((8 * K, LANES), jnp.float32)],
    )(inputs)
    return out
